# baseline (speedup 1.0000x reference)
import functools

import jax
import jax.numpy as jnp
from jax import lax
from jax.experimental import pallas as pl
from jax.experimental.pallas import tpu as pltpu
from jax.experimental.pallas import tpu_sc as plsc

N = 10000
E = 320000
NCORE = 2
NSUB = 16
NW = NCORE * NSUB
_CFG = {32: (128, 79, 4), 64: (128, 79, 4), 128: (96, 105, 2)}
ROW_STRIDE = 624
ROW_SLAB = 640


def _spmm_sc(h, src_r, dst_r, zeros, C):
  chunk, nch, nbuf = _CFG[C]
  ahead = nbuf // 2
  sfly = nbuf - ahead
  mesh = plsc.VectorSubcoreMesh(core_axis_name="c", subcore_axis_name="s")

  @functools.partial(
      pl.kernel,
      out_type=jax.ShapeDtypeStruct((NCORE, N, C), jnp.float32),
      mesh=mesh,
      compiler_params=pltpu.CompilerParams(use_tc_tiling_on_sc=False),
      scratch_types=[
          pltpu.VMEM((nch, chunk), jnp.int32),
          pltpu.VMEM((nch, chunk), jnp.int32),
          [pltpu.VMEM((chunk, C), jnp.float32)] * nbuf,
          pltpu.VMEM_SHARED((N + 8, C), jnp.float32),
          [pltpu.SemaphoreType.DMA] * nbuf,
          [pltpu.SemaphoreType.DMA] * nbuf,
      ],
  )
  def spmm(h_hbm, src_hbm, dst_hbm, z_hbm, out_hbm,
           src_v, dst_v, rows, agg_sh, gsem, ssem):
    cid = lax.axis_index("c")
    sid = lax.axis_index("s")
    wid = sid * NCORE + cid
    r0 = sid * ROW_STRIDE
    pltpu.sync_copy(z_hbm.at[pl.ds(r0, ROW_SLAB)],
                    agg_sh.at[pl.ds(r0, ROW_SLAB)])
    pltpu.sync_copy(src_hbm.at[wid], src_v)
    pltpu.sync_copy(dst_hbm.at[wid], dst_v)
    plsc.subcore_barrier()

    def wait_gather(b):
      pltpu.make_async_copy(h_hbm.at[pl.ds(0, chunk)], rows[b], gsem[b]).wait()

    def wait_scatter(b):
      pltpu.make_async_copy(rows[b], agg_sh.at[pl.ds(0, chunk)],
                            ssem[b]).wait()

    for b in range(ahead):
      pltpu.async_copy(h_hbm.at[src_v.at[b]], rows[b], gsem[b])

    @pl.loop(0, (nch + nbuf - 1) // nbuf * nbuf, step=nbuf)
    def ring(j):
      for b in range(nbuf):
        i = j + b

        @pl.when(i < nch)
        def _():
          wait_gather(b)

        @pl.when((i >= sfly) & (i < nch))
        def _():
          wait_scatter((b + ahead) % nbuf)

        @pl.when(i + ahead < nch)
        def _():
          pltpu.async_copy(h_hbm.at[src_v.at[i + ahead]],
                           rows[(b + ahead) % nbuf], gsem[(b + ahead) % nbuf])

        @pl.when(i < nch)
        def _():
          pltpu.async_copy(rows[b], agg_sh.at[dst_v.at[i]], ssem[b],
                           add=True)

    for k in range(sfly):
      wait_scatter((nch - sfly + k) % nbuf)
    plsc.subcore_barrier()
    pltpu.sync_copy(agg_sh.at[pl.ds(r0, ROW_SLAB)],
                    out_hbm.at[cid, pl.ds(r0, ROW_SLAB)])

  return spmm(h, src_r, dst_r, zeros)


def _bn(p, g, b):
  m = jnp.mean(p, axis=0, keepdims=True)
  v = jnp.mean((p - m) * (p - m), axis=0, keepdims=True)
  return (p - m) * lax.rsqrt(v + 1e-5) * g + b


def _dot(a, b):
  return jnp.dot(a, b, preferred_element_type=jnp.float32)


def _tc(body, out_shape, *args):
  return pl.pallas_call(
      body, out_shape=jax.ShapeDtypeStruct(out_shape, jnp.float32))(*args)


def _pre_stem(x, sU):
  def body(x_ref, u_ref, o_ref):
    o_ref[...] = _dot(x_ref[...], u_ref[...])
  return _tc(body, (N, sU.shape[1]), x, sU)


def _stem(x, ag, sW, sb, sg, sB):
  def body(x_ref, ag_ref, w_ref, b_ref, g_ref, bb_ref, o_ref):
    p = _dot(x_ref[...], w_ref[...]) + ag_ref[0] + ag_ref[1] + b_ref[...]
    o_ref[...] = jnp.maximum(_bn(p, g_ref[...], bb_ref[...]), 0.0)
  return _tc(body, (N, sW.shape[1]), x, ag, sW, sb, sg, sB)


def _conv_a(h, ag, W, U, b, g, B):
  def body(h_ref, ag_ref, w_ref, u_ref, b_ref, g_ref, bb_ref, o_ref):
    p = (_dot(h_ref[...], w_ref[...])
         + _dot(ag_ref[0] + ag_ref[1], u_ref[...]) + b_ref[...])
    o_ref[...] = jnp.maximum(_bn(p, g_ref[...], bb_ref[...]), 0.0)
  return _tc(body, (N, W.shape[1]), h, ag, W, U, b, g, B)


def _conv_b_res(ha, ag, hin, W, U, b, g, B, Wd, gd, Bd):
  def body(ha_ref, ag_ref, hin_ref, w_ref, u_ref, b_ref, g_ref, bb_ref,
           wd_ref, gd_ref, bd_ref, o_ref):
    p = (_dot(ha_ref[...], w_ref[...])
         + _dot(ag_ref[0] + ag_ref[1], u_ref[...]) + b_ref[...])
    hb = _bn(p, g_ref[...], bb_ref[...])
    sc = _bn(_dot(hin_ref[...], wd_ref[...]), gd_ref[...], bd_ref[...])
    o_ref[...] = jnp.maximum(hb + sc, 0.0)
  return _tc(body, (N, W.shape[1]), ha, ag, hin, W, U, b, g, B, Wd, gd, Bd)


def _final(ha, ag, hin, W, U, b, g, B, Wd, gd, Bd, lW1, lb1, lW2, lb2):
  def body(ha_ref, ag_ref, hin_ref, w_ref, u_ref, b_ref, g_ref, bb_ref,
           wd_ref, gd_ref, bd_ref, w1_ref, b1_ref, w2_ref, b2_ref, o_ref):
    p = (_dot(ha_ref[...], w_ref[...])
         + _dot(ag_ref[0] + ag_ref[1], u_ref[...]) + b_ref[...])
    hb = _bn(p, g_ref[...], bb_ref[...])
    sc = _bn(_dot(hin_ref[...], wd_ref[...]), gd_ref[...], bd_ref[...])
    h2 = jnp.maximum(hb + sc, 0.0)
    z = _dot(h2, w1_ref[...]) + b1_ref[...]
    z = jnp.where(z > 0, z, 0.1 * z)
    o_ref[...] = jax.nn.sigmoid(_dot(z, w2_ref[...]) + b2_ref[...])
  return _tc(body, (N, lW2.shape[1]), ha, ag, hin, W, U, b, g, B,
             Wd, gd, Bd, lW1, lb1, lW2, lb2)


def kernel(x, edge_index, sW, sU, sb, sg, sB,
           a1W, a1U, a1b, a1g, a1B, b1W, b1U, b1b, b1g, b1B, d1W, d1g, d1B,
           a2W, a2U, a2b, a2g, a2B, b2W, b2U, b2b, b2g, b2B, d2W, d2g, d2B,
           lW1, lb1, lW2, lb2):
  ei = edge_index.astype(jnp.int32)

  def slabs(chunk, nch):
    epad = NW * nch * chunk - E
    s = jnp.concatenate(
        [ei[0], jnp.zeros((epad,), jnp.int32)]).reshape(NW, nch, chunk)
    d = jnp.concatenate(
        [ei[1], jnp.full((epad,), N, jnp.int32)]).reshape(NW, nch, chunk)
    return s, d

  src_r, dst_r = slabs(*_CFG[32][:2])
  src_w, dst_w = slabs(*_CFG[128][:2])
  z32 = jnp.zeros((N, 32), jnp.float32)
  z64 = jnp.zeros((N, 64), jnp.float32)
  z128 = jnp.zeros((N, 128), jnp.float32)

  pre0 = _pre_stem(x, sU)
  ag0 = _spmm_sc(pre0, src_r, dst_r, z32, 32)
  h0 = _stem(x, ag0, sW, sb, sg, sB)

  ag1 = _spmm_sc(h0, src_r, dst_r, z32, 32)
  ha1 = _conv_a(h0, ag1, a1W, a1U, a1b, a1g, a1B)
  ag2 = _spmm_sc(ha1, src_r, dst_r, z64, 64)
  h1 = _conv_b_res(ha1, ag2, h0, b1W, b1U, b1b, b1g, b1B,
                   d1W, d1g, d1B)

  ag3 = _spmm_sc(h1, src_r, dst_r, z64, 64)
  ha2 = _conv_a(h1, ag3, a2W, a2U, a2b, a2g, a2B)
  ag4 = _spmm_sc(ha2, src_w, dst_w, z128, 128)
  return _final(ha2, ag4, h1, b2W, b2U, b2b, b2g, b2B,
                d2W, d2g, d2B, lW1, lb1, lW2, lb2)


# speedup vs baseline: 1.0345x; 1.0345x over previous
import functools

import jax
import jax.numpy as jnp
from jax import lax
from jax.experimental import pallas as pl
from jax.experimental.pallas import tpu as pltpu
from jax.experimental.pallas import tpu_sc as plsc

N = 10000
E = 320000
NCORE = 2
NSUB = 16
NW = NCORE * NSUB
_CFG = {32: (128, 79, 4), 64: (128, 79, 4), 128: (96, 105, 2)}
ROW_STRIDE = 624
ROW_SLAB = 640


def _spmm_sc(h, src_r, dst_r, zeros, C):
  chunk, nch, nbuf = _CFG[C]
  ahead = nbuf // 2
  sfly = nbuf - ahead
  mesh = plsc.VectorSubcoreMesh(core_axis_name="c", subcore_axis_name="s")

  @functools.partial(
      pl.kernel,
      out_type=jax.ShapeDtypeStruct((NCORE, N, C), jnp.float32),
      mesh=mesh,
      compiler_params=pltpu.CompilerParams(use_tc_tiling_on_sc=False),
      scratch_types=[
          pltpu.VMEM((nch, chunk), jnp.int32),
          pltpu.VMEM((nch, chunk), jnp.int32),
          [pltpu.VMEM((chunk, C), jnp.float32)] * nbuf,
          pltpu.VMEM_SHARED((N + 8, C), jnp.float32),
          [pltpu.SemaphoreType.DMA] * nbuf,
          [pltpu.SemaphoreType.DMA] * nbuf,
      ],
  )
  def spmm(h_hbm, src_hbm, dst_hbm, z_hbm, out_hbm,
           src_v, dst_v, rows, agg_sh, gsem, ssem):
    cid = lax.axis_index("c")
    sid = lax.axis_index("s")
    wid = sid * NCORE + cid
    r0 = sid * ROW_STRIDE
    pltpu.sync_copy(z_hbm.at[pl.ds(r0, ROW_SLAB)],
                    agg_sh.at[pl.ds(r0, ROW_SLAB)])
    pltpu.sync_copy(src_hbm.at[wid], src_v)
    pltpu.sync_copy(dst_hbm.at[wid], dst_v)
    plsc.subcore_barrier()

    def wait_gather(b):
      pltpu.make_async_copy(h_hbm.at[pl.ds(0, chunk)], rows[b], gsem[b]).wait()

    def wait_scatter(b):
      pltpu.make_async_copy(rows[b], agg_sh.at[pl.ds(0, chunk)],
                            ssem[b]).wait()

    for b in range(ahead):
      pltpu.async_copy(h_hbm.at[src_v.at[b]], rows[b], gsem[b])

    @pl.loop(0, (nch + nbuf - 1) // nbuf * nbuf, step=nbuf)
    def ring(j):
      for b in range(nbuf):
        i = j + b

        @pl.when(i < nch)
        def _():
          wait_gather(b)

        @pl.when((i >= sfly) & (i < nch))
        def _():
          wait_scatter((b + ahead) % nbuf)

        @pl.when(i + ahead < nch)
        def _():
          pltpu.async_copy(h_hbm.at[src_v.at[i + ahead]],
                           rows[(b + ahead) % nbuf], gsem[(b + ahead) % nbuf])

        @pl.when(i < nch)
        def _():
          pltpu.async_copy(rows[b], agg_sh.at[dst_v.at[i]], ssem[b],
                           add=True)

    for k in range(sfly):
      wait_scatter((nch - sfly + k) % nbuf)
    plsc.subcore_barrier()
    pltpu.sync_copy(agg_sh.at[pl.ds(r0, ROW_SLAB)],
                    out_hbm.at[cid, pl.ds(r0, ROW_SLAB)])

  return spmm(h, src_r, dst_r, zeros)


def _bn(p, g, b):
  m = jnp.mean(p, axis=0, keepdims=True)
  v = jnp.mean((p - m) * (p - m), axis=0, keepdims=True)
  return (p - m) * lax.rsqrt(v + 1e-5) * g + b


def _dot(a, b):
  return jnp.dot(a, b, preferred_element_type=jnp.float32)


def _tc(body, out_shape, *args):
  return pl.pallas_call(
      body, out_shape=jax.ShapeDtypeStruct(out_shape, jnp.float32))(*args)


def _pre_stem(x, sU):
  def body(x_ref, u_ref, o_ref):
    o_ref[...] = _dot(x_ref[...], u_ref[...])
  return _tc(body, (N, sU.shape[1]), x, sU)


def _stem(x, ag, sW, sb, sg, sB):
  def body(x_ref, ag_ref, w_ref, b_ref, g_ref, bb_ref, o_ref):
    p = _dot(x_ref[...], w_ref[...]) + ag_ref[0] + ag_ref[1] + b_ref[...]
    o_ref[...] = jnp.maximum(_bn(p, g_ref[...], bb_ref[...]), 0.0)
  return _tc(body, (N, sW.shape[1]), x, ag, sW, sb, sg, sB)


def _conv_a(h, ag, W, U, b, g, B):
  def body(h_ref, ag_ref, w_ref, u_ref, b_ref, g_ref, bb_ref, o_ref):
    p = (_dot(h_ref[...], w_ref[...])
         + _dot(ag_ref[0] + ag_ref[1], u_ref[...]) + b_ref[...])
    o_ref[...] = jnp.maximum(_bn(p, g_ref[...], bb_ref[...]), 0.0)
  return _tc(body, (N, W.shape[1]), h, ag, W, U, b, g, B)


def _conv_b_res(ha, ag, hin, W, U, b, g, B, Wd, gd, Bd):
  def body(ha_ref, ag_ref, hin_ref, w_ref, u_ref, b_ref, g_ref, bb_ref,
           wd_ref, gd_ref, bd_ref, o_ref):
    p = (_dot(ha_ref[...], w_ref[...])
         + _dot(ag_ref[0] + ag_ref[1], u_ref[...]) + b_ref[...])
    hb = _bn(p, g_ref[...], bb_ref[...])
    sc = _bn(_dot(hin_ref[...], wd_ref[...]), gd_ref[...], bd_ref[...])
    o_ref[...] = jnp.maximum(hb + sc, 0.0)
  return _tc(body, (N, W.shape[1]), ha, ag, hin, W, U, b, g, B, Wd, gd, Bd)


def _final(ha, ag, hin, W, U, b, g, B, Wd, gd, Bd, lW1, lb1, lW2, lb2):
  def body(ha_ref, ag_ref, hin_ref, w_ref, u_ref, b_ref, g_ref, bb_ref,
           wd_ref, gd_ref, bd_ref, w1_ref, b1_ref, w2_ref, b2_ref, o_ref):
    p = (_dot(ha_ref[...], w_ref[...])
         + _dot(ag_ref[0] + ag_ref[1], u_ref[...]) + b_ref[...])
    hb = _bn(p, g_ref[...], bb_ref[...])
    sc = _bn(_dot(hin_ref[...], wd_ref[...]), gd_ref[...], bd_ref[...])
    h2 = jnp.maximum(hb + sc, 0.0)
    z = _dot(h2, w1_ref[...]) + b1_ref[...]
    z = jnp.where(z > 0, z, 0.1 * z)
    o_ref[...] = jax.nn.sigmoid(_dot(z, w2_ref[...]) + b2_ref[...])
  return _tc(body, (N, lW2.shape[1]), ha, ag, hin, W, U, b, g, B,
             Wd, gd, Bd, lW1, lb1, lW2, lb2)


def kernel(x, edge_index, sW, sU, sb, sg, sB,
           a1W, a1U, a1b, a1g, a1B, b1W, b1U, b1b, b1g, b1B, d1W, d1g, d1B,
           a2W, a2U, a2b, a2g, a2B, b2W, b2U, b2b, b2g, b2B, d2W, d2g, d2B,
           lW1, lb1, lW2, lb2):
  ei = edge_index.astype(jnp.int32)

  def slabs(chunk, nch):
    ppw = nch * chunk - E // NW
    pad_s = jnp.zeros((NW, ppw), jnp.int32)
    pad_d = jnp.broadcast_to(N + (jnp.arange(ppw, dtype=jnp.int32) % 8),
                             (NW, ppw))
    s = jnp.concatenate([ei[0].reshape(NW, E // NW), pad_s], axis=1)
    d = jnp.concatenate([ei[1].reshape(NW, E // NW), pad_d], axis=1)
    return s.reshape(NW, nch, chunk), d.reshape(NW, nch, chunk)

  src_r, dst_r = slabs(*_CFG[32][:2])
  src_w, dst_w = slabs(*_CFG[128][:2])
  z32 = jnp.zeros((N, 32), jnp.float32)
  z64 = jnp.zeros((N, 64), jnp.float32)
  z128 = jnp.zeros((N, 128), jnp.float32)

  pre0 = _pre_stem(x, sU)
  ag0 = _spmm_sc(pre0, src_r, dst_r, z32, 32)
  h0 = _stem(x, ag0, sW, sb, sg, sB)

  ag1 = _spmm_sc(h0, src_r, dst_r, z32, 32)
  ha1 = _conv_a(h0, ag1, a1W, a1U, a1b, a1g, a1B)
  ag2 = _spmm_sc(ha1, src_r, dst_r, z64, 64)
  h1 = _conv_b_res(ha1, ag2, h0, b1W, b1U, b1b, b1g, b1B,
                   d1W, d1g, d1B)

  ag3 = _spmm_sc(h1, src_r, dst_r, z64, 64)
  ha2 = _conv_a(h1, ag3, a2W, a2U, a2b, a2g, a2B)
  ag4 = _spmm_sc(ha2, src_w, dst_w, z128, 128)
  return _final(ha2, ag4, h1, b2W, b2U, b2b, b2g, b2B,
                d2W, d2g, d2B, lW1, lb1, lW2, lb2)


# speedup vs baseline: 1.3885x; 1.3422x over previous
import functools

import jax
import jax.numpy as jnp
from jax import lax
from jax.experimental import pallas as pl
from jax.experimental.pallas import tpu as pltpu
from jax.experimental.pallas import tpu_sc as plsc

N = 10000
E = 320000
NCORE = 2
NSUB = 16
NW = NCORE * NSUB
_CFG = {32: (128, 79, 4, True), 64: (96, 105, 4, True), 128: (96, 105, 2, False)}
ROW_STRIDE = 624
ROW_SLAB = 640


def _spmm_sc(h, src_r, dst_r, zeros, C):
  chunk, nch, nbuf, stage = _CFG[C]
  ahead = nbuf // 2
  sfly = nbuf - ahead
  mesh = plsc.VectorSubcoreMesh(core_axis_name="c", subcore_axis_name="s")

  @functools.partial(
      pl.kernel,
      out_type=jax.ShapeDtypeStruct((NCORE, N, C), jnp.float32),
      mesh=mesh,
      compiler_params=pltpu.CompilerParams(use_tc_tiling_on_sc=False),
      scratch_types=[
          pltpu.VMEM((nch, chunk), jnp.int32),
          pltpu.VMEM((nch, chunk), jnp.int32),
          [pltpu.VMEM((chunk, C), jnp.float32)] * nbuf,
          pltpu.VMEM_SHARED((N + 8, C), jnp.float32),
          pltpu.VMEM_SHARED((N, C), jnp.float32) if stage else None,
          [pltpu.SemaphoreType.DMA] * nbuf,
          [pltpu.SemaphoreType.DMA] * nbuf,
      ],
  )
  def spmm(h_hbm, src_hbm, dst_hbm, z_hbm, out_hbm,
           src_v, dst_v, rows, agg_sh, h_sh, gsem, ssem):
    cid = lax.axis_index("c")
    sid = lax.axis_index("s")
    wid = sid * NCORE + cid
    r0 = sid * ROW_STRIDE
    pltpu.sync_copy(z_hbm.at[pl.ds(r0, ROW_SLAB)],
                    agg_sh.at[pl.ds(r0, ROW_SLAB)])
    gsrc = h_hbm
    if stage:
      pltpu.sync_copy(h_hbm.at[pl.ds(r0, ROW_SLAB)],
                      h_sh.at[pl.ds(r0, ROW_SLAB)])
      gsrc = h_sh
    pltpu.sync_copy(src_hbm.at[wid], src_v)
    pltpu.sync_copy(dst_hbm.at[wid], dst_v)
    plsc.subcore_barrier()

    def wait_gather(b):
      pltpu.make_async_copy(h_hbm.at[pl.ds(0, chunk)], rows[b], gsem[b]).wait()

    def wait_scatter(b):
      pltpu.make_async_copy(rows[b], agg_sh.at[pl.ds(0, chunk)],
                            ssem[b]).wait()

    for b in range(ahead):
      pltpu.async_copy(gsrc.at[src_v.at[b]], rows[b], gsem[b])

    @pl.loop(0, (nch + nbuf - 1) // nbuf * nbuf, step=nbuf)
    def ring(j):
      for b in range(nbuf):
        i = j + b

        @pl.when(i < nch)
        def _():
          wait_gather(b)

        @pl.when((i >= sfly) & (i < nch))
        def _():
          wait_scatter((b + ahead) % nbuf)

        @pl.when(i + ahead < nch)
        def _():
          pltpu.async_copy(gsrc.at[src_v.at[i + ahead]],
                           rows[(b + ahead) % nbuf], gsem[(b + ahead) % nbuf])

        @pl.when(i < nch)
        def _():
          pltpu.async_copy(rows[b], agg_sh.at[dst_v.at[i]], ssem[b],
                           add=True)

    for k in range(sfly):
      wait_scatter((nch - sfly + k) % nbuf)
    plsc.subcore_barrier()
    pltpu.sync_copy(agg_sh.at[pl.ds(r0, ROW_SLAB)],
                    out_hbm.at[cid, pl.ds(r0, ROW_SLAB)])

  return spmm(h, src_r, dst_r, zeros)


def _bn(p, g, b):
  m = jnp.mean(p, axis=0, keepdims=True)
  v = jnp.mean((p - m) * (p - m), axis=0, keepdims=True)
  return (p - m) * lax.rsqrt(v + 1e-5) * g + b


def _dot(a, b):
  return jnp.dot(a, b, preferred_element_type=jnp.float32)


def _tc(body, out_shape, *args):
  return pl.pallas_call(
      body, out_shape=jax.ShapeDtypeStruct(out_shape, jnp.float32))(*args)


def _pre_stem(x, sU):
  def body(x_ref, u_ref, o_ref):
    o_ref[...] = _dot(x_ref[...], u_ref[...])
  return _tc(body, (N, sU.shape[1]), x, sU)


def _stem(x, ag, sW, sb, sg, sB):
  def body(x_ref, ag_ref, w_ref, b_ref, g_ref, bb_ref, o_ref):
    p = _dot(x_ref[...], w_ref[...]) + ag_ref[0] + ag_ref[1] + b_ref[...]
    o_ref[...] = jnp.maximum(_bn(p, g_ref[...], bb_ref[...]), 0.0)
  return _tc(body, (N, sW.shape[1]), x, ag, sW, sb, sg, sB)


def _conv_a(h, ag, W, U, b, g, B):
  def body(h_ref, ag_ref, w_ref, u_ref, b_ref, g_ref, bb_ref, o_ref):
    p = (_dot(h_ref[...], w_ref[...])
         + _dot(ag_ref[0] + ag_ref[1], u_ref[...]) + b_ref[...])
    o_ref[...] = jnp.maximum(_bn(p, g_ref[...], bb_ref[...]), 0.0)
  return _tc(body, (N, W.shape[1]), h, ag, W, U, b, g, B)


def _conv_b_res(ha, ag, hin, W, U, b, g, B, Wd, gd, Bd):
  def body(ha_ref, ag_ref, hin_ref, w_ref, u_ref, b_ref, g_ref, bb_ref,
           wd_ref, gd_ref, bd_ref, o_ref):
    p = (_dot(ha_ref[...], w_ref[...])
         + _dot(ag_ref[0] + ag_ref[1], u_ref[...]) + b_ref[...])
    hb = _bn(p, g_ref[...], bb_ref[...])
    sc = _bn(_dot(hin_ref[...], wd_ref[...]), gd_ref[...], bd_ref[...])
    o_ref[...] = jnp.maximum(hb + sc, 0.0)
  return _tc(body, (N, W.shape[1]), ha, ag, hin, W, U, b, g, B, Wd, gd, Bd)


def _final(ha, ag, hin, W, U, b, g, B, Wd, gd, Bd, lW1, lb1, lW2, lb2):
  def body(ha_ref, ag_ref, hin_ref, w_ref, u_ref, b_ref, g_ref, bb_ref,
           wd_ref, gd_ref, bd_ref, w1_ref, b1_ref, w2_ref, b2_ref, o_ref):
    p = (_dot(ha_ref[...], w_ref[...])
         + _dot(ag_ref[0] + ag_ref[1], u_ref[...]) + b_ref[...])
    hb = _bn(p, g_ref[...], bb_ref[...])
    sc = _bn(_dot(hin_ref[...], wd_ref[...]), gd_ref[...], bd_ref[...])
    h2 = jnp.maximum(hb + sc, 0.0)
    z = _dot(h2, w1_ref[...]) + b1_ref[...]
    z = jnp.where(z > 0, z, 0.1 * z)
    o_ref[...] = jax.nn.sigmoid(_dot(z, w2_ref[...]) + b2_ref[...])
  return _tc(body, (N, lW2.shape[1]), ha, ag, hin, W, U, b, g, B,
             Wd, gd, Bd, lW1, lb1, lW2, lb2)


def kernel(x, edge_index, sW, sU, sb, sg, sB,
           a1W, a1U, a1b, a1g, a1B, b1W, b1U, b1b, b1g, b1B, d1W, d1g, d1B,
           a2W, a2U, a2b, a2g, a2B, b2W, b2U, b2b, b2g, b2B, d2W, d2g, d2B,
           lW1, lb1, lW2, lb2):
  ei = edge_index.astype(jnp.int32)

  def slabs(chunk, nch):
    ppw = nch * chunk - E // NW
    pad_s = jnp.zeros((NW, ppw), jnp.int32)
    pad_d = jnp.broadcast_to(N + (jnp.arange(ppw, dtype=jnp.int32) % 8),
                             (NW, ppw))
    s = jnp.concatenate([ei[0].reshape(NW, E // NW), pad_s], axis=1)
    d = jnp.concatenate([ei[1].reshape(NW, E // NW), pad_d], axis=1)
    return s.reshape(NW, nch, chunk), d.reshape(NW, nch, chunk)

  src_r, dst_r = slabs(*_CFG[32][:2])
  src_w, dst_w = slabs(*_CFG[128][:2])
  z32 = jnp.zeros((N, 32), jnp.float32)
  z64 = jnp.zeros((N, 64), jnp.float32)
  z128 = jnp.zeros((N, 128), jnp.float32)

  pre0 = _pre_stem(x, sU)
  ag0 = _spmm_sc(pre0, src_r, dst_r, z32, 32)
  h0 = _stem(x, ag0, sW, sb, sg, sB)

  ag1 = _spmm_sc(h0, src_r, dst_r, z32, 32)
  ha1 = _conv_a(h0, ag1, a1W, a1U, a1b, a1g, a1B)
  ag2 = _spmm_sc(ha1, src_w, dst_w, z64, 64)
  h1 = _conv_b_res(ha1, ag2, h0, b1W, b1U, b1b, b1g, b1B,
                   d1W, d1g, d1B)

  ag3 = _spmm_sc(h1, src_w, dst_w, z64, 64)
  ha2 = _conv_a(h1, ag3, a2W, a2U, a2b, a2g, a2B)
  ag4 = _spmm_sc(ha2, src_w, dst_w, z128, 128)
  return _final(ha2, ag4, h1, b2W, b2U, b2b, b2g, b2B,
                d2W, d2g, d2B, lW1, lb1, lW2, lb2)


# speedup vs baseline: 1.6114x; 1.1605x over previous
import functools

import jax
import jax.numpy as jnp
from jax import lax
from jax.experimental import pallas as pl
from jax.experimental.pallas import tpu as pltpu
from jax.experimental.pallas import tpu_sc as plsc

N = 10000
E = 320000
NCORE = 2
NSUB = 16
NW = NCORE * NSUB
_CFG = {32: (128, 79, 4), 64: (96, 105, 4)}
ROW_STRIDE = 624
ROW_SLAB = 640


def _spmm_sc(h, src_r, dst_r, zeros, C):
  chunk, nch, nbuf = _CFG[C]
  ahead = nbuf // 2
  sfly = nbuf - ahead
  mesh = plsc.VectorSubcoreMesh(core_axis_name="c", subcore_axis_name="s")

  @functools.partial(
      pl.kernel,
      out_type=jax.ShapeDtypeStruct((NCORE, N, C), jnp.float32),
      mesh=mesh,
      compiler_params=pltpu.CompilerParams(use_tc_tiling_on_sc=False),
      scratch_types=[
          pltpu.VMEM((nch, chunk), jnp.int32),
          pltpu.VMEM((nch, chunk), jnp.int32),
          [pltpu.VMEM((chunk, C), jnp.float32)] * nbuf,
          pltpu.VMEM_SHARED((N + 8, C), jnp.float32),
          pltpu.VMEM_SHARED((N, C), jnp.float32),
          [pltpu.SemaphoreType.DMA] * nbuf,
          [pltpu.SemaphoreType.DMA] * nbuf,
      ],
  )
  def spmm(h_hbm, src_hbm, dst_hbm, z_hbm, out_hbm,
           src_v, dst_v, rows, agg_sh, h_sh, gsem, ssem):
    cid = lax.axis_index("c")
    sid = lax.axis_index("s")
    wid = sid * NCORE + cid
    r0 = sid * ROW_STRIDE
    pltpu.sync_copy(z_hbm.at[pl.ds(r0, ROW_SLAB)],
                    agg_sh.at[pl.ds(r0, ROW_SLAB)])
    gsrc = h_sh
    pltpu.sync_copy(h_hbm.at[pl.ds(r0, ROW_SLAB)],
                    h_sh.at[pl.ds(r0, ROW_SLAB)])
    pltpu.sync_copy(src_hbm.at[wid], src_v)
    pltpu.sync_copy(dst_hbm.at[wid], dst_v)
    plsc.subcore_barrier()

    def wait_gather(b):
      pltpu.make_async_copy(h_hbm.at[pl.ds(0, chunk)], rows[b], gsem[b]).wait()

    def wait_scatter(b):
      pltpu.make_async_copy(rows[b], agg_sh.at[pl.ds(0, chunk)],
                            ssem[b]).wait()

    for b in range(ahead):
      pltpu.async_copy(gsrc.at[src_v.at[b]], rows[b], gsem[b])

    @pl.loop(0, (nch + nbuf - 1) // nbuf * nbuf, step=nbuf)
    def ring(j):
      for b in range(nbuf):
        i = j + b

        @pl.when(i < nch)
        def _():
          wait_gather(b)

        @pl.when((i >= sfly) & (i < nch))
        def _():
          wait_scatter((b + ahead) % nbuf)

        @pl.when(i + ahead < nch)
        def _():
          pltpu.async_copy(gsrc.at[src_v.at[i + ahead]],
                           rows[(b + ahead) % nbuf], gsem[(b + ahead) % nbuf])

        @pl.when(i < nch)
        def _():
          pltpu.async_copy(rows[b], agg_sh.at[dst_v.at[i]], ssem[b],
                           add=True)

    for k in range(sfly):
      wait_scatter((nch - sfly + k) % nbuf)
    plsc.subcore_barrier()
    pltpu.sync_copy(agg_sh.at[pl.ds(r0, ROW_SLAB)],
                    out_hbm.at[cid, pl.ds(r0, ROW_SLAB)])

  return spmm(h, src_r, dst_r, zeros)


def _bn(p, g, b):
  m = jnp.mean(p, axis=0, keepdims=True)
  v = jnp.mean((p - m) * (p - m), axis=0, keepdims=True)
  return (p - m) * lax.rsqrt(v + 1e-5) * g + b


def _dot(a, b):
  return jnp.dot(a, b, preferred_element_type=jnp.float32)


def _tc(body, out_shape, *args):
  return pl.pallas_call(
      body, out_shape=jax.ShapeDtypeStruct(out_shape, jnp.float32))(*args)


def _pre_stem(x, sU):
  def body(x_ref, u_ref, o_ref):
    o_ref[...] = _dot(x_ref[...], u_ref[...])
  return _tc(body, (N, sU.shape[1]), x, sU)


def _stem(x, ag, sW, sb, sg, sB):
  def body(x_ref, ag_ref, w_ref, b_ref, g_ref, bb_ref, o_ref):
    p = _dot(x_ref[...], w_ref[...]) + ag_ref[0] + ag_ref[1] + b_ref[...]
    o_ref[...] = jnp.maximum(_bn(p, g_ref[...], bb_ref[...]), 0.0)
  return _tc(body, (N, sW.shape[1]), x, ag, sW, sb, sg, sB)


def _conv_a(h, ag, W, U, b, g, B, split=False):
  cout = W.shape[1]

  def body(h_ref, ag_ref, w_ref, u_ref, b_ref, g_ref, bb_ref, *o_refs):
    p = (_dot(h_ref[...], w_ref[...])
         + _dot(ag_ref[0] + ag_ref[1], u_ref[...]) + b_ref[...])
    r = jnp.maximum(_bn(p, g_ref[...], bb_ref[...]), 0.0)
    if split:
      o_refs[0][...] = r[:, :cout // 2]
      o_refs[1][...] = r[:, cout // 2:]
    else:
      o_refs[0][...] = r

  out_shape = (jax.ShapeDtypeStruct((N, cout // 2), jnp.float32),) * 2 \
      if split else jax.ShapeDtypeStruct((N, cout), jnp.float32)
  return pl.pallas_call(body, out_shape=out_shape)(h, ag, W, U, b, g, B)


def _conv_b_res(ha, ag, hin, W, U, b, g, B, Wd, gd, Bd):
  def body(ha_ref, ag_ref, hin_ref, w_ref, u_ref, b_ref, g_ref, bb_ref,
           wd_ref, gd_ref, bd_ref, o_ref):
    p = (_dot(ha_ref[...], w_ref[...])
         + _dot(ag_ref[0] + ag_ref[1], u_ref[...]) + b_ref[...])
    hb = _bn(p, g_ref[...], bb_ref[...])
    sc = _bn(_dot(hin_ref[...], wd_ref[...]), gd_ref[...], bd_ref[...])
    o_ref[...] = jnp.maximum(hb + sc, 0.0)
  return _tc(body, (N, W.shape[1]), ha, ag, hin, W, U, b, g, B, Wd, gd, Bd)


def _final(haa, hab, aga, agb, hin, W, U, b, g, B, Wd, gd, Bd,
           lW1, lb1, lW2, lb2):
  def body(haa_ref, hab_ref, aga_ref, agb_ref, hin_ref, w_ref, u_ref, b_ref,
           g_ref, bb_ref, wd_ref, gd_ref, bd_ref, w1_ref, b1_ref, w2_ref,
           b2_ref, o_ref):
    u = u_ref[...]
    w = w_ref[...]
    half = u.shape[0] // 2
    p = (_dot(haa_ref[...], w[:half]) + _dot(hab_ref[...], w[half:])
         + _dot(aga_ref[0] + aga_ref[1], u[:half])
         + _dot(agb_ref[0] + agb_ref[1], u[half:]) + b_ref[...])
    hb = _bn(p, g_ref[...], bb_ref[...])
    sc = _bn(_dot(hin_ref[...], wd_ref[...]), gd_ref[...], bd_ref[...])
    h2 = jnp.maximum(hb + sc, 0.0)
    z = _dot(h2, w1_ref[...]) + b1_ref[...]
    z = jnp.where(z > 0, z, 0.1 * z)
    o_ref[...] = jax.nn.sigmoid(_dot(z, w2_ref[...]) + b2_ref[...])
  return _tc(body, (N, lW2.shape[1]), haa, hab, aga, agb, hin, W, U, b, g, B,
             Wd, gd, Bd, lW1, lb1, lW2, lb2)


def kernel(x, edge_index, sW, sU, sb, sg, sB,
           a1W, a1U, a1b, a1g, a1B, b1W, b1U, b1b, b1g, b1B, d1W, d1g, d1B,
           a2W, a2U, a2b, a2g, a2B, b2W, b2U, b2b, b2g, b2B, d2W, d2g, d2B,
           lW1, lb1, lW2, lb2):
  ei = edge_index.astype(jnp.int32)

  def slabs(chunk, nch):
    ppw = nch * chunk - E // NW
    pad_s = jnp.zeros((NW, ppw), jnp.int32)
    pad_d = jnp.broadcast_to(N + (jnp.arange(ppw, dtype=jnp.int32) % 8),
                             (NW, ppw))
    s = jnp.concatenate([ei[0].reshape(NW, E // NW), pad_s], axis=1)
    d = jnp.concatenate([ei[1].reshape(NW, E // NW), pad_d], axis=1)
    return s.reshape(NW, nch, chunk), d.reshape(NW, nch, chunk)

  src_r, dst_r = slabs(*_CFG[32][:2])
  src_w, dst_w = slabs(*_CFG[64][:2])
  z32 = jnp.zeros((N, 32), jnp.float32)
  z64 = jnp.zeros((N, 64), jnp.float32)

  pre0 = _pre_stem(x, sU)
  ag0 = _spmm_sc(pre0, src_r, dst_r, z32, 32)
  h0 = _stem(x, ag0, sW, sb, sg, sB)

  ag1 = _spmm_sc(h0, src_r, dst_r, z32, 32)
  ha1 = _conv_a(h0, ag1, a1W, a1U, a1b, a1g, a1B)
  ag2 = _spmm_sc(ha1, src_w, dst_w, z64, 64)
  h1 = _conv_b_res(ha1, ag2, h0, b1W, b1U, b1b, b1g, b1B,
                   d1W, d1g, d1B)

  ag3 = _spmm_sc(h1, src_w, dst_w, z64, 64)
  ha2a, ha2b = _conv_a(h1, ag3, a2W, a2U, a2b, a2g, a2B,
                       split=True)
  ag4a = _spmm_sc(ha2a, src_w, dst_w, z64, 64)
  ag4b = _spmm_sc(ha2b, src_w, dst_w, z64, 64)
  return _final(ha2a, ha2b, ag4a, ag4b, h1, b2W, b2U, b2b, b2g, b2B,
                d2W, d2g, d2B, lW1, lb1, lW2, lb2)
